# baseline (speedup 1.0000x reference)
import functools

import jax
import jax.numpy as jnp
from jax import lax
from jax.experimental import pallas as pl
from jax.experimental.pallas import tpu as pltpu
from jax.experimental.pallas import tpu_sc as plsc

_E = 64
_TOPK = 2
_D = 768
_DFF = 1536
_N = 4096

_TM = 64
_NT = (_N * _TOPK) // _TM + _E
_NP = _NT * _TM
_DFB = 512
_NDF = _DFF // _DFB

_TB = 256


def _router_body(x_ref, gw_ref, gb_ref, i1_ref, i2_ref, s1_ref, s2_ref):
    logits = jnp.dot(x_ref[...], gw_ref[...],
                     preferred_element_type=jnp.float32) + gb_ref[...]
    cols = lax.broadcasted_iota(jnp.int32, logits.shape, 1)
    m1 = jnp.max(logits, axis=1, keepdims=True)
    i1 = jnp.min(jnp.where(logits == m1, cols, _E), axis=1, keepdims=True)
    masked = jnp.where(cols == i1, -jnp.inf, logits)
    m2 = jnp.max(masked, axis=1, keepdims=True)
    i2 = jnp.min(jnp.where(masked == m2, cols, _E), axis=1, keepdims=True)
    z = jnp.exp(m2 - m1)
    denom = 1.0 + z
    i1_ref[...] = i1
    i2_ref[...] = i2
    s1_ref[...] = 1.0 / denom
    s2_ref[...] = z / denom


def _route(x, gate_w, gate_b):
    n_blk = _N // _TB
    outs = pl.pallas_call(
        _router_body,
        grid=(n_blk,),
        in_specs=[
            pl.BlockSpec((_TB, _D), lambda t: (t, 0)),
            pl.BlockSpec((_D, _E), lambda t: (0, 0)),
            pl.BlockSpec((1, _E), lambda t: (0, 0)),
        ],
        out_specs=[
            pl.BlockSpec((_TB, 1), lambda t: (t, 0)),
            pl.BlockSpec((_TB, 1), lambda t: (t, 0)),
            pl.BlockSpec((_TB, 1), lambda t: (t, 0)),
            pl.BlockSpec((_TB, 1), lambda t: (t, 0)),
        ],
        out_shape=[
            jax.ShapeDtypeStruct((_N, 1), jnp.int32),
            jax.ShapeDtypeStruct((_N, 1), jnp.int32),
            jax.ShapeDtypeStruct((_N, 1), jnp.float32),
            jax.ShapeDtypeStruct((_N, 1), jnp.float32),
        ],
    )(x, gate_w, gate_b.reshape(1, _E))
    i1, i2, s1, s2 = outs
    top_i = jnp.concatenate([i1, i2], axis=1)
    score = jnp.concatenate([s1, s2], axis=1)
    return top_i, score


def _mlp_body(eot_ref, live_ref, x_ref, w1_ref, w2_ref, b1_ref, b2_ref,
              sc_ref, out_ref):
    t = pl.program_id(0)

    @pl.when(live_ref[t] != 0)
    def _compute():
        a = (jnp.dot(x_ref[...], w1_ref[0],
                     preferred_element_type=jnp.float32) + b1_ref[0])
        h = 0.5 * a * (1.0 + lax.erf(a * 0.7071067811865476))
        out_ref[...] = (jnp.dot(h, w2_ref[0],
                                preferred_element_type=jnp.float32)
                        + b2_ref[0]) * sc_ref[...]


def _expert_mlp(eot, live, x_padded, w1, w2, b1, b2, score_padded):
    grid_spec = pltpu.PrefetchScalarGridSpec(
        num_scalar_prefetch=2,
        grid=(_NT,),
        in_specs=[
            pl.BlockSpec((_TM, _D), lambda t, eot, live: (t, 0)),
            pl.BlockSpec((1, _D, _DFF), lambda t, eot, live: (eot[t], 0, 0)),
            pl.BlockSpec((1, _DFF, _D), lambda t, eot, live: (eot[t], 0, 0)),
            pl.BlockSpec((1, 1, _DFF), lambda t, eot, live: (eot[t], 0, 0)),
            pl.BlockSpec((1, 1, _D), lambda t, eot, live: (eot[t], 0, 0)),
            pl.BlockSpec((_TM, 1), lambda t, eot, live: (t, 0)),
        ],
        out_specs=pl.BlockSpec((_TM, _D), lambda t, eot, live: (t, 0)),
    )
    return pl.pallas_call(
        _mlp_body,
        grid_spec=grid_spec,
        out_shape=jax.ShapeDtypeStruct((_NP, _D), jnp.float32),
        compiler_params=pltpu.CompilerParams(
            dimension_semantics=("arbitrary",)),
    )(eot, live, x_padded, w1, w2, b1.reshape(_E, 1, _DFF),
      b2.reshape(_E, 1, _D), score_padded)


_NC = 2
_NS = 16
_NW = _NC * _NS

_GCHUNK = 64


def _sc_gather_rows(table, idx, n_rows):
    rows_per_w = n_rows // _NW
    n_chunks = rows_per_w // _GCHUNK
    d = table.shape[1]
    mesh = plsc.VectorSubcoreMesh(core_axis_name="c", subcore_axis_name="s")

    @functools.partial(
        pl.kernel, mesh=mesh,
        out_type=jax.ShapeDtypeStruct((n_rows, d), jnp.float32),
        scratch_types=[
            pltpu.VMEM((_GCHUNK,), jnp.int32),
            pltpu.VMEM((_GCHUNK, d), jnp.float32),
            pltpu.SemaphoreType.DMA,
        ],
    )
    def _gather(table_hbm, idx_hbm, out_hbm, idx_v, rows_v, sem):
        wid = lax.axis_index("s") * _NC + lax.axis_index("c")
        base = wid * rows_per_w

        def body(i, _):
            off = base + i * _GCHUNK
            pltpu.sync_copy(idx_hbm.at[pl.ds(off, _GCHUNK)], idx_v)
            pltpu.async_copy(table_hbm.at[idx_v], rows_v, sem).wait()
            pltpu.sync_copy(rows_v, out_hbm.at[pl.ds(off, _GCHUNK)])
            return _

        lax.fori_loop(0, n_chunks, body, 0)

    return _gather(table, idx)


_CCHUNK = 32


def _sc_combine(y, l0, l1):
    toks_per_w = _N // _NW
    n_chunks = toks_per_w // _CCHUNK
    mesh = plsc.VectorSubcoreMesh(core_axis_name="c", subcore_axis_name="s")

    @functools.partial(
        pl.kernel, mesh=mesh,
        out_type=jax.ShapeDtypeStruct((_N, _D), jnp.float32),
        scratch_types=[
            pltpu.VMEM((_CCHUNK,), jnp.int32),
            pltpu.VMEM((_CCHUNK,), jnp.int32),
            pltpu.VMEM((_CCHUNK, _D), jnp.float32),
            pltpu.VMEM((_CCHUNK, _D), jnp.float32),
            pltpu.SemaphoreType.DMA,
            pltpu.SemaphoreType.DMA,
        ],
    )
    def _combine(y_hbm, l0_hbm, l1_hbm, out_hbm, i0_v, i1_v, a_v, b_v,
                 sem0, sem1):
        wid = lax.axis_index("s") * _NC + lax.axis_index("c")
        base = wid * toks_per_w

        def body(i, _):
            off = base + i * _CCHUNK
            pltpu.sync_copy(l0_hbm.at[pl.ds(off, _CCHUNK)], i0_v)
            pltpu.sync_copy(l1_hbm.at[pl.ds(off, _CCHUNK)], i1_v)
            cp0 = pltpu.async_copy(y_hbm.at[i0_v], a_v, sem0)
            cp1 = pltpu.async_copy(y_hbm.at[i1_v], b_v, sem1)
            cp0.wait()
            cp1.wait()

            def row(r, _):
                for j in range(_D // 16):
                    sl = pl.ds(j * 16, 16)
                    a_v[r, sl] = a_v[r, sl] + b_v[r, sl]
                return _

            lax.fori_loop(0, _CCHUNK, row, 0)
            pltpu.sync_copy(a_v, out_hbm.at[pl.ds(off, _CCHUNK)])
            return _

        lax.fori_loop(0, n_chunks, body, 0)

    return _combine(y, l0, l1)


def kernel(x, gate_w, gate_b, w1, b1, w2, b2):
    top_i, score = _route(x, gate_w, gate_b)

    nc = _N * _TOPK
    flat_e = top_i.reshape(-1)
    flat_s = score.reshape(-1)
    order = jnp.argsort(flat_e)
    sorted_e = flat_e[order]
    counts = jnp.bincount(flat_e, length=_E)
    tiles = (counts + _TM - 1) // _TM
    tile_ends = jnp.cumsum(tiles)
    pstart = _TM * (tile_ends - tiles)
    starts = jnp.cumsum(counts) - counts
    pos_sorted = jnp.arange(nc, dtype=jnp.int32) - starts[sorted_e]
    ploc_sorted = (pstart[sorted_e] + pos_sorted).astype(jnp.int32)
    tok_sorted = (order // _TOPK).astype(jnp.int32)
    tok_padded = (jnp.arange(_NP, dtype=jnp.int32) % _N).at[
        ploc_sorted].set(tok_sorted)
    ploc = jnp.zeros((nc,), jnp.int32).at[order].set(ploc_sorted)
    score_padded = jnp.zeros((_NP,), jnp.float32).at[ploc_sorted].set(
        flat_s[order])
    tidx = jnp.arange(_NT)
    eot = jnp.clip(
        jnp.searchsorted(tile_ends, tidx, side='right'),
        0, _E - 1).astype(jnp.int32)
    live = (tidx < tile_ends[-1]).astype(jnp.int32)

    x_padded = _sc_gather_rows(x, tok_padded, _NP)
    y = _expert_mlp(eot, live, x_padded, w1, w2, b1, b2,
                    score_padded.reshape(_NP, 1))
    ploc2 = ploc.reshape(_N, _TOPK)
    return _sc_combine(y, ploc2[:, 0], ploc2[:, 1])


# speedup vs baseline: 1.6914x; 1.6914x over previous
import functools

import jax
import jax.numpy as jnp
from jax import lax
from jax.experimental import pallas as pl
from jax.experimental.pallas import tpu as pltpu
from jax.experimental.pallas import tpu_sc as plsc

_E = 64
_TOPK = 2
_D = 768
_DFF = 1536
_N = 4096

_TM = 128
_NT = (_N * _TOPK) // _TM + _E
_NP = _NT * _TM

_TB = 256
_NB = _N // _TB

_NC = 2
_NS = 16
_NW = _NC * _NS


def _router_body(x_ref, gw_ref, gb_ref, i1_ref, i2_ref, s1_ref, s2_ref,
                 r1_ref, r2_ref, hist_ref):
    logits = jnp.dot(x_ref[...], gw_ref[...],
                     preferred_element_type=jnp.float32) + gb_ref[...]
    cols = lax.broadcasted_iota(jnp.int32, logits.shape, 1)
    m1 = jnp.max(logits, axis=1, keepdims=True)
    i1 = jnp.min(jnp.where(logits == m1, cols, _E), axis=1, keepdims=True)
    masked = jnp.where(cols == i1, -jnp.inf, logits)
    m2 = jnp.max(masked, axis=1, keepdims=True)
    i2 = jnp.min(jnp.where(masked == m2, cols, _E), axis=1, keepdims=True)
    z = jnp.exp(m2 - m1)
    denom = 1.0 + z
    i1_ref[...] = i1
    i2_ref[...] = i2
    s1_ref[...] = 1.0 / denom
    s2_ref[...] = z / denom

    o1 = (cols == i1).astype(jnp.float32)
    o2 = (cols == i2).astype(jnp.float32)
    s = o1 + o2
    rows_i = lax.broadcasted_iota(jnp.int32, (_TB, _TB), 0)
    cols_i = lax.broadcasted_iota(jnp.int32, (_TB, _TB), 1)
    ltri = (cols_i < rows_i).astype(jnp.float32)
    c = jnp.dot(ltri, s, preferred_element_type=jnp.float32)
    r1_ref[...] = jnp.sum(c * o1, axis=1, keepdims=True).astype(jnp.int32)
    r2_ref[...] = jnp.sum(c * o2, axis=1, keepdims=True).astype(jnp.int32)
    hist_ref[...] = jnp.sum(s, axis=0, keepdims=True)[None].astype(jnp.int32)


def _route(x, gate_w, gate_b):
    outs = pl.pallas_call(
        _router_body,
        grid=(_NB,),
        in_specs=[
            pl.BlockSpec((_TB, _D), lambda t: (t, 0)),
            pl.BlockSpec((_D, _E), lambda t: (0, 0)),
            pl.BlockSpec((1, _E), lambda t: (0, 0)),
        ],
        out_specs=[pl.BlockSpec((_TB, 1), lambda t: (t, 0))] * 6
        + [pl.BlockSpec((1, 1, _E), lambda t: (t, 0, 0))],
        out_shape=[
            jax.ShapeDtypeStruct((_N, 1), jnp.int32),
            jax.ShapeDtypeStruct((_N, 1), jnp.int32),
            jax.ShapeDtypeStruct((_N, 1), jnp.float32),
            jax.ShapeDtypeStruct((_N, 1), jnp.float32),
            jax.ShapeDtypeStruct((_N, 1), jnp.int32),
            jax.ShapeDtypeStruct((_N, 1), jnp.int32),
            jax.ShapeDtypeStruct((_NB, 1, _E), jnp.int32),
        ],
    )(x, gate_w, gate_b.reshape(1, _E))
    return outs


def _slot_body(i1_ref, i2_ref, r1_ref, r2_ref, base_ref, s1_ref, s2_ref,
               p1_ref, p2_ref, s1b_ref, s2b_ref):
    cols = lax.broadcasted_iota(jnp.int32, (_TB, _E), 1)
    base = base_ref[0]
    o1 = cols == i1_ref[...]
    o2 = cols == i2_ref[...]
    b1 = jnp.sum(jnp.where(o1, base, 0), axis=1, keepdims=True)
    b2 = jnp.sum(jnp.where(o2, base, 0), axis=1, keepdims=True)
    p1_ref[...] = b1 + r1_ref[...]
    p2_ref[...] = b2 + r2_ref[...]
    s1b_ref[...] = jnp.broadcast_to(s1_ref[...], (_TB, 16))
    s2b_ref[...] = jnp.broadcast_to(s2_ref[...], (_TB, 16))


def _slots(i1, i2, r1, r2, baseb, s1, s2):
    return pl.pallas_call(
        _slot_body,
        grid=(_NB,),
        in_specs=[pl.BlockSpec((_TB, 1), lambda t: (t, 0))] * 4
        + [pl.BlockSpec((1, 1, _E), lambda t: (t, 0, 0))]
        + [pl.BlockSpec((_TB, 1), lambda t: (t, 0))] * 2,
        out_specs=[pl.BlockSpec((_TB, 1), lambda t: (t, 0))] * 2
        + [pl.BlockSpec((_TB, 16), lambda t: (t, 0))] * 2,
        out_shape=[
            jax.ShapeDtypeStruct((_N, 1), jnp.int32),
            jax.ShapeDtypeStruct((_N, 1), jnp.int32),
            jax.ShapeDtypeStruct((_N, 16), jnp.float32),
            jax.ShapeDtypeStruct((_N, 16), jnp.float32),
        ],
    )(i1, i2, r1, r2, baseb, s1, s2)


def _mlp_body(eot_ref, live_ref, x_ref, w1_ref, w2_ref, b1_ref, b2_ref,
              out_ref):
    t = pl.program_id(0)

    @pl.when(live_ref[t] != 0)
    def _compute():
        a = (jnp.dot(x_ref[...], w1_ref[0],
                     preferred_element_type=jnp.float32) + b1_ref[0])
        h = 0.5 * a * (1.0 + lax.erf(a * 0.7071067811865476))
        out_ref[...] = (jnp.dot(h, w2_ref[0],
                                preferred_element_type=jnp.float32)
                        + b2_ref[0])


def _expert_mlp(eot, live, x_padded, w1, w2, b1, b2):
    grid_spec = pltpu.PrefetchScalarGridSpec(
        num_scalar_prefetch=2,
        grid=(_NT,),
        in_specs=[
            pl.BlockSpec((_TM, _D), lambda t, eot, live: (t, 0)),
            pl.BlockSpec((1, _D, _DFF), lambda t, eot, live: (eot[t], 0, 0)),
            pl.BlockSpec((1, _DFF, _D), lambda t, eot, live: (eot[t], 0, 0)),
            pl.BlockSpec((1, 1, _DFF), lambda t, eot, live: (eot[t], 0, 0)),
            pl.BlockSpec((1, 1, _D), lambda t, eot, live: (eot[t], 0, 0)),
        ],
        out_specs=pl.BlockSpec((_TM, _D), lambda t, eot, live: (t, 0)),
    )
    return pl.pallas_call(
        _mlp_body,
        grid_spec=grid_spec,
        out_shape=jax.ShapeDtypeStruct((_NP, _D), jnp.float32),
        compiler_params=pltpu.CompilerParams(
            dimension_semantics=("arbitrary",)),
    )(eot, live, x_padded, w1, w2, b1.reshape(_E, 1, _DFF),
      b2.reshape(_E, 1, _D))


_DCHUNK = 32


def _sc_dispatch(x, p1, p2):
    toks_per_w = _N // _NW
    n_chunks = toks_per_w // _DCHUNK
    mesh = plsc.VectorSubcoreMesh(core_axis_name="c", subcore_axis_name="s")

    @functools.partial(
        pl.kernel, mesh=mesh,
        out_type=jax.ShapeDtypeStruct((_NP, _D), jnp.float32),
        scratch_types=[
            pltpu.VMEM((_DCHUNK,), jnp.int32),
            pltpu.VMEM((_DCHUNK,), jnp.int32),
            pltpu.VMEM((_DCHUNK, _D), jnp.float32),
            pltpu.SemaphoreType.DMA,
            pltpu.SemaphoreType.DMA,
        ],
    )
    def _dispatch(x_hbm, p1_hbm, p2_hbm, out_hbm, i0_v, i1_v, xbuf, s0, s1):
        wid = lax.axis_index("s") * _NC + lax.axis_index("c")
        base = wid * toks_per_w

        def body(i, _):
            off = base + i * _DCHUNK
            pltpu.sync_copy(x_hbm.at[pl.ds(off, _DCHUNK)], xbuf)
            pltpu.sync_copy(p1_hbm.at[pl.ds(off, _DCHUNK)], i0_v)
            pltpu.sync_copy(p2_hbm.at[pl.ds(off, _DCHUNK)], i1_v)
            cp0 = pltpu.async_copy(xbuf, out_hbm.at[i0_v], s0)
            cp1 = pltpu.async_copy(xbuf, out_hbm.at[i1_v], s1)
            cp0.wait()
            cp1.wait()
            return _

        lax.fori_loop(0, n_chunks, body, 0)

    return _dispatch(x, p1, p2)


def _sc_combine(y, p1, p2, sc1, sc2):
    toks_per_w = _N // _NW
    n_chunks = toks_per_w // _DCHUNK
    mesh = plsc.VectorSubcoreMesh(core_axis_name="c", subcore_axis_name="s")

    @functools.partial(
        pl.kernel, mesh=mesh,
        out_type=jax.ShapeDtypeStruct((_N, _D), jnp.float32),
        scratch_types=[
            pltpu.VMEM((_DCHUNK,), jnp.int32),
            pltpu.VMEM((_DCHUNK,), jnp.int32),
            pltpu.VMEM((_DCHUNK, 16), jnp.float32),
            pltpu.VMEM((_DCHUNK, 16), jnp.float32),
            pltpu.VMEM((_DCHUNK, _D), jnp.float32),
            pltpu.VMEM((_DCHUNK, _D), jnp.float32),
            pltpu.SemaphoreType.DMA,
            pltpu.SemaphoreType.DMA,
        ],
    )
    def _combine(y_hbm, p1_hbm, p2_hbm, sc1_hbm, sc2_hbm, out_hbm,
                 i0_v, i1_v, s0_v, s1_v, a_v, b_v, sem0, sem1):
        wid = lax.axis_index("s") * _NC + lax.axis_index("c")
        base = wid * toks_per_w

        def body(i, _):
            off = base + i * _DCHUNK
            pltpu.sync_copy(p1_hbm.at[pl.ds(off, _DCHUNK)], i0_v)
            pltpu.sync_copy(p2_hbm.at[pl.ds(off, _DCHUNK)], i1_v)
            pltpu.sync_copy(sc1_hbm.at[pl.ds(off, _DCHUNK)], s0_v)
            pltpu.sync_copy(sc2_hbm.at[pl.ds(off, _DCHUNK)], s1_v)
            cp0 = pltpu.async_copy(y_hbm.at[i0_v], a_v, sem0)
            cp1 = pltpu.async_copy(y_hbm.at[i1_v], b_v, sem1)
            cp0.wait()
            cp1.wait()

            def row(r, _):
                sa = s0_v[r, pl.ds(0, 16)]
                sb = s1_v[r, pl.ds(0, 16)]
                for j in range(_D // 16):
                    sl = pl.ds(j * 16, 16)
                    a_v[r, sl] = sa * a_v[r, sl] + sb * b_v[r, sl]
                return _

            lax.fori_loop(0, _DCHUNK, row, 0)
            pltpu.sync_copy(a_v, out_hbm.at[pl.ds(off, _DCHUNK)])
            return _

        lax.fori_loop(0, n_chunks, body, 0)

    return _combine(y, p1, p2, sc1, sc2)


def kernel(x, gate_w, gate_b, w1, b1, w2, b2):
    i1, i2, s1, s2, r1, r2, hist = _route(x, gate_w, gate_b)

    h = hist.reshape(_NB, _E)
    bb = jnp.cumsum(h, axis=0) - h
    counts = h.sum(axis=0)
    tiles = (counts + _TM - 1) // _TM
    tile_ends = jnp.cumsum(tiles)
    pstart = _TM * (tile_ends - tiles)
    baseb = (bb + pstart[None, :]).astype(jnp.int32).reshape(_NB, 1, _E)
    tidx = jnp.arange(_NT)
    eot = jnp.clip(jnp.searchsorted(tile_ends, tidx, side='right'),
                   0, _E - 1).astype(jnp.int32)
    live = (tidx < tile_ends[-1]).astype(jnp.int32)

    p1, p2, s1b, s2b = _slots(i1, i2, r1, r2, baseb, s1, s2)
    p1 = p1.reshape(_N)
    p2 = p2.reshape(_N)

    x_padded = _sc_dispatch(x, p1, p2)
    y = _expert_mlp(eot, live, x_padded, w1, w2, b1, b2)
    return _sc_combine(y, p1, p2, s1b, s2b)


# speedup vs baseline: 1.8164x; 1.0739x over previous
import functools

import jax
import jax.numpy as jnp
from jax import lax
from jax.experimental import pallas as pl
from jax.experimental.pallas import tpu as pltpu
from jax.experimental.pallas import tpu_sc as plsc

_E = 64
_TOPK = 2
_D = 768
_DFF = 1536
_N = 4096

_TM = 128
_NT = (_N * _TOPK) // _TM + _E
_NP = _NT * _TM

_TB = 256
_NB = _N // _TB

_NC = 2
_NS = 16
_NW = _NC * _NS


def _router_body(x_ref, gw_ref, gb_ref, i1_ref, i2_ref, s1_ref, s2_ref,
                 r1_ref, r2_ref, hist_ref):
    logits = jnp.dot(x_ref[...], gw_ref[...],
                     preferred_element_type=jnp.float32) + gb_ref[...]
    cols = lax.broadcasted_iota(jnp.int32, logits.shape, 1)
    m1 = jnp.max(logits, axis=1, keepdims=True)
    i1 = jnp.min(jnp.where(logits == m1, cols, _E), axis=1, keepdims=True)
    masked = jnp.where(cols == i1, -jnp.inf, logits)
    m2 = jnp.max(masked, axis=1, keepdims=True)
    i2 = jnp.min(jnp.where(masked == m2, cols, _E), axis=1, keepdims=True)
    z = jnp.exp(m2 - m1)
    denom = 1.0 + z
    i1_ref[...] = i1
    i2_ref[...] = i2
    s1_ref[...] = 1.0 / denom
    s2_ref[...] = z / denom

    o1 = (cols == i1).astype(jnp.float32)
    o2 = (cols == i2).astype(jnp.float32)
    s = o1 + o2
    rows_i = lax.broadcasted_iota(jnp.int32, (_TB, _TB), 0)
    cols_i = lax.broadcasted_iota(jnp.int32, (_TB, _TB), 1)
    ltri = (cols_i < rows_i).astype(jnp.float32)
    c = jnp.dot(ltri, s, preferred_element_type=jnp.float32)
    r1_ref[...] = jnp.sum(c * o1, axis=1, keepdims=True).astype(jnp.int32)
    r2_ref[...] = jnp.sum(c * o2, axis=1, keepdims=True).astype(jnp.int32)
    hist_ref[...] = jnp.sum(s, axis=0, keepdims=True)[None].astype(jnp.int32)


def _route(x, gate_w, gate_b):
    outs = pl.pallas_call(
        _router_body,
        grid=(_NB,),
        in_specs=[
            pl.BlockSpec((_TB, _D), lambda t: (t, 0)),
            pl.BlockSpec((_D, _E), lambda t: (0, 0)),
            pl.BlockSpec((1, _E), lambda t: (0, 0)),
        ],
        out_specs=[pl.BlockSpec((_TB, 1), lambda t: (t, 0))] * 6
        + [pl.BlockSpec((1, 1, _E), lambda t: (t, 0, 0))],
        out_shape=[
            jax.ShapeDtypeStruct((_N, 1), jnp.int32),
            jax.ShapeDtypeStruct((_N, 1), jnp.int32),
            jax.ShapeDtypeStruct((_N, 1), jnp.float32),
            jax.ShapeDtypeStruct((_N, 1), jnp.float32),
            jax.ShapeDtypeStruct((_N, 1), jnp.int32),
            jax.ShapeDtypeStruct((_N, 1), jnp.int32),
            jax.ShapeDtypeStruct((_NB, 1, _E), jnp.int32),
        ],
    )(x, gate_w, gate_b.reshape(1, _E))
    return outs


def _slot_body(i1_ref, i2_ref, r1_ref, r2_ref, base_ref, s1_ref, s2_ref,
               p1_ref, p2_ref, s1b_ref, s2b_ref):
    cols = lax.broadcasted_iota(jnp.int32, (_TB, _E), 1)
    base = base_ref[0]
    o1 = cols == i1_ref[...]
    o2 = cols == i2_ref[...]
    b1 = jnp.sum(jnp.where(o1, base, 0), axis=1, keepdims=True)
    b2 = jnp.sum(jnp.where(o2, base, 0), axis=1, keepdims=True)
    p1_ref[...] = b1 + r1_ref[...]
    p2_ref[...] = b2 + r2_ref[...]
    s1b_ref[...] = jnp.broadcast_to(s1_ref[...], (_TB, 16))
    s2b_ref[...] = jnp.broadcast_to(s2_ref[...], (_TB, 16))


def _slots(i1, i2, r1, r2, baseb, s1, s2):
    return pl.pallas_call(
        _slot_body,
        grid=(_NB,),
        in_specs=[pl.BlockSpec((_TB, 1), lambda t: (t, 0))] * 4
        + [pl.BlockSpec((1, 1, _E), lambda t: (t, 0, 0))]
        + [pl.BlockSpec((_TB, 1), lambda t: (t, 0))] * 2,
        out_specs=[pl.BlockSpec((_TB, 1), lambda t: (t, 0))] * 2
        + [pl.BlockSpec((_TB, 16), lambda t: (t, 0))] * 2,
        out_shape=[
            jax.ShapeDtypeStruct((_N, 1), jnp.int32),
            jax.ShapeDtypeStruct((_N, 1), jnp.int32),
            jax.ShapeDtypeStruct((_N, 16), jnp.float32),
            jax.ShapeDtypeStruct((_N, 16), jnp.float32),
        ],
    )(i1, i2, r1, r2, baseb, s1, s2)


def _mlp_body(eot_ref, live_ref, start_ref, run_ref, runs_e_ref, nr_ref,
              x_ref, w1_hbm, w2_hbm, b1_ref, b2_ref, out_ref,
              w1b, w2b, sem1, sem2):
    t = pl.program_id(0)
    r = run_ref[t]

    def _start_fetch(rf):
        slot = lax.rem(rf, 3)
        e = runs_e_ref[rf]
        pltpu.make_async_copy(w1_hbm.at[e], w1b.at[slot],
                              sem1.at[slot]).start()
        pltpu.make_async_copy(w2_hbm.at[e], w2b.at[slot],
                              sem2.at[slot]).start()

    @pl.when(t == 0)
    def _prologue():
        _start_fetch(0)

        @pl.when(nr_ref[0] > 1)
        def _p1():
            _start_fetch(1)

        @pl.when(nr_ref[0] > 2)
        def _p2():
            _start_fetch(2)

    @pl.when((start_ref[t] != 0) & (t > 0) & (r + 2 < nr_ref[0]))
    def _fetch_ahead():
        _start_fetch(r + 2)

    @pl.when(start_ref[t] != 0)
    def _wait():
        slot = lax.rem(r, 3)
        e = runs_e_ref[r]
        pltpu.make_async_copy(w1_hbm.at[e], w1b.at[slot],
                              sem1.at[slot]).wait()
        pltpu.make_async_copy(w2_hbm.at[e], w2b.at[slot],
                              sem2.at[slot]).wait()

    @pl.when(live_ref[t] != 0)
    def _compute():
        slot = lax.rem(r, 3)
        a = (jnp.dot(x_ref[...], w1b[slot],
                     preferred_element_type=jnp.float32) + b1_ref[0])
        h = 0.5 * a * (1.0 + lax.erf(a * 0.7071067811865476))
        out_ref[...] = (jnp.dot(h, w2b[slot],
                                preferred_element_type=jnp.float32)
                        + b2_ref[0])


def _expert_mlp(eot, live, is_start, run_of, runs_e, n_runs,
                x_padded, w1, w2, b1, b2):
    grid_spec = pltpu.PrefetchScalarGridSpec(
        num_scalar_prefetch=6,
        grid=(_NT,),
        in_specs=[
            pl.BlockSpec((_TM, _D), lambda t, *_: (t, 0)),
            pl.BlockSpec(memory_space=pltpu.MemorySpace.HBM),
            pl.BlockSpec(memory_space=pltpu.MemorySpace.HBM),
            pl.BlockSpec((1, 1, _DFF), lambda t, eot, *_: (eot[t], 0, 0)),
            pl.BlockSpec((1, 1, _D), lambda t, eot, *_: (eot[t], 0, 0)),
        ],
        out_specs=pl.BlockSpec((_TM, _D), lambda t, *_: (t, 0)),
        scratch_shapes=[
            pltpu.VMEM((3, _D, _DFF), jnp.float32),
            pltpu.VMEM((3, _DFF, _D), jnp.float32),
            pltpu.SemaphoreType.DMA((3,)),
            pltpu.SemaphoreType.DMA((3,)),
        ],
    )
    return pl.pallas_call(
        _mlp_body,
        grid_spec=grid_spec,
        out_shape=jax.ShapeDtypeStruct((_NP, _D), jnp.float32),
        compiler_params=pltpu.CompilerParams(
            dimension_semantics=("arbitrary",)),
    )(eot, live, is_start, run_of, runs_e, n_runs, x_padded, w1, w2,
      b1.reshape(_E, 1, _DFF), b2.reshape(_E, 1, _D))


_DCHUNK = 32


def _sc_dispatch(x, p1, p2):
    toks_per_w = _N // _NW
    n_chunks = toks_per_w // _DCHUNK
    mesh = plsc.VectorSubcoreMesh(core_axis_name="c", subcore_axis_name="s")

    @functools.partial(
        pl.kernel, mesh=mesh,
        out_type=jax.ShapeDtypeStruct((_NP, _D), jnp.float32),
        scratch_types=[
            pltpu.VMEM((_DCHUNK,), jnp.int32),
            pltpu.VMEM((_DCHUNK,), jnp.int32),
            pltpu.VMEM((_DCHUNK, _D), jnp.float32),
            pltpu.SemaphoreType.DMA,
            pltpu.SemaphoreType.DMA,
        ],
    )
    def _dispatch(x_hbm, p1_hbm, p2_hbm, out_hbm, i0_v, i1_v, xbuf, s0, s1):
        wid = lax.axis_index("s") * _NC + lax.axis_index("c")
        base = wid * toks_per_w

        def body(i, _):
            off = base + i * _DCHUNK
            pltpu.sync_copy(x_hbm.at[pl.ds(off, _DCHUNK)], xbuf)
            pltpu.sync_copy(p1_hbm.at[pl.ds(off, _DCHUNK)], i0_v)
            pltpu.sync_copy(p2_hbm.at[pl.ds(off, _DCHUNK)], i1_v)
            cp0 = pltpu.async_copy(xbuf, out_hbm.at[i0_v], s0)
            cp1 = pltpu.async_copy(xbuf, out_hbm.at[i1_v], s1)
            cp0.wait()
            cp1.wait()
            return _

        lax.fori_loop(0, n_chunks, body, 0)

    return _dispatch(x, p1, p2)


def _sc_combine(y, p1, p2, sc1, sc2):
    toks_per_w = _N // _NW
    n_chunks = toks_per_w // _DCHUNK
    mesh = plsc.VectorSubcoreMesh(core_axis_name="c", subcore_axis_name="s")

    @functools.partial(
        pl.kernel, mesh=mesh,
        out_type=jax.ShapeDtypeStruct((_N, _D), jnp.float32),
        scratch_types=[
            pltpu.VMEM((_DCHUNK,), jnp.int32),
            pltpu.VMEM((_DCHUNK,), jnp.int32),
            pltpu.VMEM((_DCHUNK, 16), jnp.float32),
            pltpu.VMEM((_DCHUNK, 16), jnp.float32),
            pltpu.VMEM((_DCHUNK, _D), jnp.float32),
            pltpu.VMEM((_DCHUNK, _D), jnp.float32),
            pltpu.SemaphoreType.DMA,
            pltpu.SemaphoreType.DMA,
        ],
    )
    def _combine(y_hbm, p1_hbm, p2_hbm, sc1_hbm, sc2_hbm, out_hbm,
                 i0_v, i1_v, s0_v, s1_v, a_v, b_v, sem0, sem1):
        wid = lax.axis_index("s") * _NC + lax.axis_index("c")
        base = wid * toks_per_w

        def body(i, _):
            off = base + i * _DCHUNK
            pltpu.sync_copy(p1_hbm.at[pl.ds(off, _DCHUNK)], i0_v)
            pltpu.sync_copy(p2_hbm.at[pl.ds(off, _DCHUNK)], i1_v)
            pltpu.sync_copy(sc1_hbm.at[pl.ds(off, _DCHUNK)], s0_v)
            pltpu.sync_copy(sc2_hbm.at[pl.ds(off, _DCHUNK)], s1_v)
            cp0 = pltpu.async_copy(y_hbm.at[i0_v], a_v, sem0)
            cp1 = pltpu.async_copy(y_hbm.at[i1_v], b_v, sem1)
            cp0.wait()
            cp1.wait()

            def row(r, _):
                sa = s0_v[r, pl.ds(0, 16)]
                sb = s1_v[r, pl.ds(0, 16)]
                for j in range(_D // 16):
                    sl = pl.ds(j * 16, 16)
                    a_v[r, sl] = sa * a_v[r, sl] + sb * b_v[r, sl]
                return _

            lax.fori_loop(0, _DCHUNK, row, 0)
            pltpu.sync_copy(a_v, out_hbm.at[pl.ds(off, _DCHUNK)])
            return _

        lax.fori_loop(0, n_chunks, body, 0)

    return _combine(y, p1, p2, sc1, sc2)


def kernel(x, gate_w, gate_b, w1, b1, w2, b2):
    i1, i2, s1, s2, r1, r2, hist = _route(x, gate_w, gate_b)

    h = hist.reshape(_NB, _E)
    bb = jnp.cumsum(h, axis=0) - h
    counts = h.sum(axis=0)
    tiles = (counts + _TM - 1) // _TM
    tile_ends = jnp.cumsum(tiles)
    pstart = _TM * (tile_ends - tiles)
    baseb = (bb + pstart[None, :]).astype(jnp.int32).reshape(_NB, 1, _E)
    tidx = jnp.arange(_NT)
    eot = jnp.clip(jnp.searchsorted(tile_ends, tidx, side='right'),
                   0, _E - 1).astype(jnp.int32)
    live = (tidx < tile_ends[-1]).astype(jnp.int32)
    is_start = jnp.concatenate(
        [jnp.ones((1,), jnp.int32),
         (eot[1:] != eot[:-1]).astype(jnp.int32)])
    run_of = jnp.cumsum(is_start) - 1
    runs_e = jnp.zeros((_NT,), jnp.int32).at[run_of].set(eot)
    n_runs = run_of[-1:] + 1
    run_of = run_of.astype(jnp.int32)
    n_runs = n_runs.astype(jnp.int32)

    p1, p2, s1b, s2b = _slots(i1, i2, r1, r2, baseb, s1, s2)
    p1 = p1.reshape(_N)
    p2 = p2.reshape(_N)

    x_padded = _sc_dispatch(x, p1, p2)
    y = _expert_mlp(eot, live, is_start, run_of, runs_e, n_runs,
                    x_padded, w1, w2, b1, b2)
    return _sc_combine(y, p1, p2, s1b, s2b)


# speedup vs baseline: 1.9019x; 1.0471x over previous
import functools

import jax
import jax.numpy as jnp
from jax import lax
from jax.experimental import pallas as pl
from jax.experimental.pallas import tpu as pltpu
from jax.experimental.pallas import tpu_sc as plsc

_E = 64
_TOPK = 2
_D = 768
_DFF = 1536
_N = 4096

_TM = 128
_NT = (_N * _TOPK) // _TM + _E
_NP = _NT * _TM

_TB = 256
_NB = _N // _TB

_NC = 2
_NS = 16
_NW = _NC * _NS


def _router_body(x_ref, gw_ref, gb_ref, i1_ref, i2_ref, s1_ref, s2_ref,
                 r1_ref, r2_ref, hist_ref):
    logits = jnp.dot(x_ref[...], gw_ref[...],
                     preferred_element_type=jnp.float32) + gb_ref[...]
    cols = lax.broadcasted_iota(jnp.int32, logits.shape, 1)
    m1 = jnp.max(logits, axis=1, keepdims=True)
    i1 = jnp.min(jnp.where(logits == m1, cols, _E), axis=1, keepdims=True)
    masked = jnp.where(cols == i1, -jnp.inf, logits)
    m2 = jnp.max(masked, axis=1, keepdims=True)
    i2 = jnp.min(jnp.where(masked == m2, cols, _E), axis=1, keepdims=True)
    z = jnp.exp(m2 - m1)
    denom = 1.0 + z
    i1_ref[...] = i1
    i2_ref[...] = i2
    s1_ref[...] = 1.0 / denom
    s2_ref[...] = z / denom

    o1 = (cols == i1).astype(jnp.float32)
    o2 = (cols == i2).astype(jnp.float32)
    s = o1 + o2
    rows_i = lax.broadcasted_iota(jnp.int32, (_TB, _TB), 0)
    cols_i = lax.broadcasted_iota(jnp.int32, (_TB, _TB), 1)
    ltri = (cols_i < rows_i).astype(jnp.float32)
    c = jnp.dot(ltri, s, preferred_element_type=jnp.float32)
    r1_ref[...] = jnp.sum(c * o1, axis=1, keepdims=True).astype(jnp.int32)
    r2_ref[...] = jnp.sum(c * o2, axis=1, keepdims=True).astype(jnp.int32)
    hist_ref[...] = jnp.sum(s, axis=0, keepdims=True)[None].astype(jnp.int32)


def _route(x, gate_w, gate_b):
    outs = pl.pallas_call(
        _router_body,
        grid=(_NB,),
        in_specs=[
            pl.BlockSpec((_TB, _D), lambda t: (t, 0)),
            pl.BlockSpec((_D, _E), lambda t: (0, 0)),
            pl.BlockSpec((1, _E), lambda t: (0, 0)),
        ],
        out_specs=[pl.BlockSpec((_TB, 1), lambda t: (t, 0))] * 6
        + [pl.BlockSpec((1, 1, _E), lambda t: (t, 0, 0))],
        out_shape=[
            jax.ShapeDtypeStruct((_N, 1), jnp.int32),
            jax.ShapeDtypeStruct((_N, 1), jnp.int32),
            jax.ShapeDtypeStruct((_N, 1), jnp.float32),
            jax.ShapeDtypeStruct((_N, 1), jnp.float32),
            jax.ShapeDtypeStruct((_N, 1), jnp.int32),
            jax.ShapeDtypeStruct((_N, 1), jnp.int32),
            jax.ShapeDtypeStruct((_NB, 1, _E), jnp.int32),
        ],
    )(x, gate_w, gate_b.reshape(1, _E))
    return outs


def _slot_body(i1_ref, i2_ref, r1_ref, r2_ref, base_ref, s1_ref, s2_ref,
               p1_ref, p2_ref, s1b_ref, s2b_ref):
    cols = lax.broadcasted_iota(jnp.int32, (_TB, _E), 1)
    base = base_ref[0]
    o1 = cols == i1_ref[...]
    o2 = cols == i2_ref[...]
    b1 = jnp.sum(jnp.where(o1, base, 0), axis=1, keepdims=True)
    b2 = jnp.sum(jnp.where(o2, base, 0), axis=1, keepdims=True)
    p1_ref[...] = b1 + r1_ref[...]
    p2_ref[...] = b2 + r2_ref[...]
    s1b_ref[...] = jnp.broadcast_to(s1_ref[...], (_TB, 16))
    s2b_ref[...] = jnp.broadcast_to(s2_ref[...], (_TB, 16))


def _slots(i1, i2, r1, r2, baseb, s1, s2):
    return pl.pallas_call(
        _slot_body,
        grid=(_NB,),
        in_specs=[pl.BlockSpec((_TB, 1), lambda t: (t, 0))] * 4
        + [pl.BlockSpec((1, 1, _E), lambda t: (t, 0, 0))]
        + [pl.BlockSpec((_TB, 1), lambda t: (t, 0))] * 2,
        out_specs=[pl.BlockSpec((_TB, 1), lambda t: (t, 0))] * 2
        + [pl.BlockSpec((_TB, 16), lambda t: (t, 0))] * 2,
        out_shape=[
            jax.ShapeDtypeStruct((_N, 1), jnp.int32),
            jax.ShapeDtypeStruct((_N, 1), jnp.int32),
            jax.ShapeDtypeStruct((_N, 16), jnp.float32),
            jax.ShapeDtypeStruct((_N, 16), jnp.float32),
        ],
    )(i1, i2, r1, r2, baseb, s1, s2)


def _mlp_body(eot_ref, live_ref, start_ref, run_ref, runs_e_ref, nr_ref,
              x_ref, w1_hbm, w2_hbm, b1_ref, b2_ref, out_ref,
              w1b, w2b, sem1, sem2):
    t = pl.program_id(0)
    r = run_ref[t]

    def _start_fetch(rf):
        slot = lax.rem(rf, 3)
        e = runs_e_ref[rf]
        pltpu.make_async_copy(w1_hbm.at[e], w1b.at[slot],
                              sem1.at[slot]).start()
        pltpu.make_async_copy(w2_hbm.at[e], w2b.at[slot],
                              sem2.at[slot]).start()

    @pl.when(t == 0)
    def _prologue():
        _start_fetch(0)

        @pl.when(nr_ref[0] > 1)
        def _p1():
            _start_fetch(1)

        @pl.when(nr_ref[0] > 2)
        def _p2():
            _start_fetch(2)

    @pl.when((start_ref[t] != 0) & (t > 0) & (r + 2 < nr_ref[0]))
    def _fetch_ahead():
        _start_fetch(r + 2)

    @pl.when(start_ref[t] != 0)
    def _wait():
        slot = lax.rem(r, 3)
        e = runs_e_ref[r]
        pltpu.make_async_copy(w1_hbm.at[e], w1b.at[slot],
                              sem1.at[slot]).wait()
        pltpu.make_async_copy(w2_hbm.at[e], w2b.at[slot],
                              sem2.at[slot]).wait()

    @pl.when(live_ref[t] != 0)
    def _compute():
        slot = lax.rem(r, 3)
        a = (jnp.dot(x_ref[...], w1b[slot],
                     preferred_element_type=jnp.float32) + b1_ref[0])
        h = 0.5 * a * (1.0 + lax.erf(a * 0.7071067811865476))
        out_ref[...] = (jnp.dot(h, w2b[slot],
                                preferred_element_type=jnp.float32)
                        + b2_ref[0])


def _expert_mlp(eot, live, is_start, run_of, runs_e, n_runs,
                x_padded, w1, w2, b1, b2):
    grid_spec = pltpu.PrefetchScalarGridSpec(
        num_scalar_prefetch=6,
        grid=(_NT,),
        in_specs=[
            pl.BlockSpec((_TM, _D), lambda t, *_: (t, 0)),
            pl.BlockSpec(memory_space=pltpu.MemorySpace.HBM),
            pl.BlockSpec(memory_space=pltpu.MemorySpace.HBM),
            pl.BlockSpec((1, 1, _DFF), lambda t, eot, *_: (eot[t], 0, 0)),
            pl.BlockSpec((1, 1, _D), lambda t, eot, *_: (eot[t], 0, 0)),
        ],
        out_specs=pl.BlockSpec((_TM, _D), lambda t, *_: (t, 0)),
        scratch_shapes=[
            pltpu.VMEM((3, _D, _DFF), jnp.float32),
            pltpu.VMEM((3, _DFF, _D), jnp.float32),
            pltpu.SemaphoreType.DMA((3,)),
            pltpu.SemaphoreType.DMA((3,)),
        ],
    )
    return pl.pallas_call(
        _mlp_body,
        grid_spec=grid_spec,
        out_shape=jax.ShapeDtypeStruct((_NP, _D), jnp.float32),
        compiler_params=pltpu.CompilerParams(
            dimension_semantics=("arbitrary",)),
    )(eot, live, is_start, run_of, runs_e, n_runs, x_padded, w1, w2,
      b1.reshape(_E, 1, _DFF), b2.reshape(_E, 1, _D))


_DCHUNK = 32
_NCH = (_N // _NW) // _DCHUNK
_CCH = 16
_NCC = (_N // _NW) // _CCH


def _sc_dispatch(x, p1, p2):
    toks_per_w = _N // _NW
    mesh = plsc.VectorSubcoreMesh(core_axis_name="c", subcore_axis_name="s")

    @functools.partial(
        pl.kernel, mesh=mesh,
        out_type=jax.ShapeDtypeStruct((_NP, _D), jnp.float32),
        scratch_types=[
            pltpu.VMEM((_NCH, _DCHUNK), jnp.int32),
            pltpu.VMEM((_NCH, _DCHUNK), jnp.int32),
            pltpu.VMEM((2, _DCHUNK, _D), jnp.float32),
            pltpu.SemaphoreType.DMA((2,)),
            pltpu.SemaphoreType.DMA,
            pltpu.SemaphoreType.DMA,
        ],
    )
    def _dispatch(x_hbm, p1_hbm, p2_hbm, out_hbm, i0_v, i1_v, xbuf,
                  sx, s0, s1):
        wid = lax.axis_index("s") * _NC + lax.axis_index("c")
        base = wid * toks_per_w
        pltpu.sync_copy(p1_hbm.at[wid], i0_v)
        pltpu.sync_copy(p2_hbm.at[wid], i1_v)

        def _xread(i, slot):
            return pltpu.make_async_copy(
                x_hbm.at[pl.ds(base + i * _DCHUNK, _DCHUNK)],
                xbuf.at[slot], sx.at[slot])

        _xread(0, 0).start()

        def body(i, _):
            slot = lax.rem(i, 2)
            _xread(i, slot).wait()

            @pl.when(i + 1 < _NCH)
            def _ahead():
                _xread(i + 1, 1 - slot).start()

            cp0 = pltpu.async_copy(xbuf.at[slot], out_hbm.at[i0_v.at[i]], s0)
            cp1 = pltpu.async_copy(xbuf.at[slot], out_hbm.at[i1_v.at[i]], s1)
            cp0.wait()
            cp1.wait()
            return _

        lax.fori_loop(0, _NCH, body, 0)

    return _dispatch(x, p1.reshape(_NW, _NCH, _DCHUNK),
                     p2.reshape(_NW, _NCH, _DCHUNK))


def _sc_combine(y, p1, p2, sc1, sc2):
    toks_per_w = _N // _NW
    mesh = plsc.VectorSubcoreMesh(core_axis_name="c", subcore_axis_name="s")

    @functools.partial(
        pl.kernel, mesh=mesh,
        out_type=jax.ShapeDtypeStruct((_N, _D), jnp.float32),
        scratch_types=[
            pltpu.VMEM((_NCC, _CCH), jnp.int32),
            pltpu.VMEM((_NCC, _CCH), jnp.int32),
            pltpu.VMEM((_NCC, _CCH, 16), jnp.float32),
            pltpu.VMEM((_NCC, _CCH, 16), jnp.float32),
            pltpu.VMEM((2, _CCH, _D), jnp.float32),
            pltpu.VMEM((2, _CCH, _D), jnp.float32),
            pltpu.SemaphoreType.DMA((2,)),
            pltpu.SemaphoreType.DMA((2,)),
        ],
    )
    def _combine(y_hbm, p1_hbm, p2_hbm, sc1_hbm, sc2_hbm, out_hbm,
                 i0_v, i1_v, s0_v, s1_v, a_v, b_v, sem0, sem1):
        wid = lax.axis_index("s") * _NC + lax.axis_index("c")
        base = wid * toks_per_w
        pltpu.sync_copy(p1_hbm.at[wid], i0_v)
        pltpu.sync_copy(p2_hbm.at[wid], i1_v)
        pltpu.sync_copy(sc1_hbm.at[wid], s0_v)
        pltpu.sync_copy(sc2_hbm.at[wid], s1_v)

        def _gathers(i, slot):
            return (pltpu.make_async_copy(y_hbm.at[i0_v.at[i]],
                                          a_v.at[slot], sem0.at[slot]),
                    pltpu.make_async_copy(y_hbm.at[i1_v.at[i]],
                                          b_v.at[slot], sem1.at[slot]))

        g0, g1 = _gathers(0, 0)
        g0.start()
        g1.start()

        def body(i, _):
            slot = lax.rem(i, 2)
            g0, g1 = _gathers(i, slot)
            g0.wait()
            g1.wait()

            @pl.when(i + 1 < _NCC)
            def _ahead():
                n0, n1 = _gathers(i + 1, 1 - slot)
                n0.start()
                n1.start()

            def row(r, _):
                sa = s0_v[i, r, pl.ds(0, 16)]
                sb = s1_v[i, r, pl.ds(0, 16)]
                for j in range(_D // 16):
                    sl = pl.ds(j * 16, 16)
                    a_v[slot, r, sl] = (sa * a_v[slot, r, sl]
                                        + sb * b_v[slot, r, sl])
                return _

            lax.fori_loop(0, _CCH, row, 0)
            pltpu.sync_copy(
                a_v.at[slot],
                out_hbm.at[pl.ds(base + i * _CCH, _CCH)])
            return _

        lax.fori_loop(0, _NCC, body, 0)

    return _combine(y, p1.reshape(_NW, _NCC, _CCH),
                    p2.reshape(_NW, _NCC, _CCH),
                    sc1.reshape(_NW, _NCC, _CCH, 16),
                    sc2.reshape(_NW, _NCC, _CCH, 16))


def kernel(x, gate_w, gate_b, w1, b1, w2, b2):
    i1, i2, s1, s2, r1, r2, hist = _route(x, gate_w, gate_b)

    h = hist.reshape(_NB, _E)
    bb = jnp.cumsum(h, axis=0) - h
    counts = h.sum(axis=0)
    tiles = (counts + _TM - 1) // _TM
    tile_ends = jnp.cumsum(tiles)
    pstart = _TM * (tile_ends - tiles)
    baseb = (bb + pstart[None, :]).astype(jnp.int32).reshape(_NB, 1, _E)
    tidx = jnp.arange(_NT)
    eot = jnp.clip(jnp.searchsorted(tile_ends, tidx, side='right'),
                   0, _E - 1).astype(jnp.int32)
    live = (tidx < tile_ends[-1]).astype(jnp.int32)
    is_start = jnp.concatenate(
        [jnp.ones((1,), jnp.int32),
         (eot[1:] != eot[:-1]).astype(jnp.int32)])
    run_of = jnp.cumsum(is_start) - 1
    runs_e = jnp.zeros((_NT,), jnp.int32).at[run_of].set(eot)
    n_runs = run_of[-1:] + 1
    run_of = run_of.astype(jnp.int32)
    n_runs = n_runs.astype(jnp.int32)

    p1, p2, s1b, s2b = _slots(i1, i2, r1, r2, baseb, s1, s2)
    p1 = p1.reshape(_N)
    p2 = p2.reshape(_N)

    x_padded = _sc_dispatch(x, p1, p2)
    y = _expert_mlp(eot, live, is_start, run_of, runs_e, n_runs,
                    x_padded, w1, w2, b1, b2)
    return _sc_combine(y, p1, p2, s1b, s2b)


# speedup vs baseline: 1.9633x; 1.0323x over previous
import functools

import jax
import jax.numpy as jnp
from jax import lax
from jax.experimental import pallas as pl
from jax.experimental.pallas import tpu as pltpu
from jax.experimental.pallas import tpu_sc as plsc

_E = 64
_TOPK = 2
_D = 768
_DFF = 1536
_N = 4096

_TM = 128
_NT = (_N * _TOPK) // _TM + _E
_NP = _NT * _TM

_TB = 512
_NB = _N // _TB

_NC = 2
_NS = 16
_NW = _NC * _NS


def _router_body(x_ref, gw_ref, gb_ref, i1_ref, i2_ref, s1_ref, s2_ref,
                 r1_ref, r2_ref, hist_ref):
    logits = jnp.dot(x_ref[...], gw_ref[...],
                     preferred_element_type=jnp.float32) + gb_ref[...]
    cols = lax.broadcasted_iota(jnp.int32, logits.shape, 1)
    m1 = jnp.max(logits, axis=1, keepdims=True)
    i1 = jnp.min(jnp.where(logits == m1, cols, _E), axis=1, keepdims=True)
    masked = jnp.where(cols == i1, -jnp.inf, logits)
    m2 = jnp.max(masked, axis=1, keepdims=True)
    i2 = jnp.min(jnp.where(masked == m2, cols, _E), axis=1, keepdims=True)
    z = jnp.exp(m2 - m1)
    denom = 1.0 + z
    i1_ref[...] = i1
    i2_ref[...] = i2
    s1_ref[...] = 1.0 / denom
    s2_ref[...] = z / denom

    o1 = (cols == i1).astype(jnp.float32)
    o2 = (cols == i2).astype(jnp.float32)
    s = o1 + o2
    rows_i = lax.broadcasted_iota(jnp.int32, (_TB, _TB), 0)
    cols_i = lax.broadcasted_iota(jnp.int32, (_TB, _TB), 1)
    ltri = (cols_i < rows_i).astype(jnp.float32)
    c = jnp.dot(ltri, s, preferred_element_type=jnp.float32)
    r1_ref[...] = jnp.sum(c * o1, axis=1, keepdims=True).astype(jnp.int32)
    r2_ref[...] = jnp.sum(c * o2, axis=1, keepdims=True).astype(jnp.int32)
    hist_ref[...] = jnp.sum(s, axis=0, keepdims=True)[None].astype(jnp.int32)


def _route(x, gate_w, gate_b):
    outs = pl.pallas_call(
        _router_body,
        grid=(_NB,),
        in_specs=[
            pl.BlockSpec((_TB, _D), lambda t: (t, 0)),
            pl.BlockSpec((_D, _E), lambda t: (0, 0)),
            pl.BlockSpec((1, _E), lambda t: (0, 0)),
        ],
        out_specs=[pl.BlockSpec((_TB, 1), lambda t: (t, 0))] * 6
        + [pl.BlockSpec((1, 1, _E), lambda t: (t, 0, 0))],
        out_shape=[
            jax.ShapeDtypeStruct((_N, 1), jnp.int32),
            jax.ShapeDtypeStruct((_N, 1), jnp.int32),
            jax.ShapeDtypeStruct((_N, 1), jnp.float32),
            jax.ShapeDtypeStruct((_N, 1), jnp.float32),
            jax.ShapeDtypeStruct((_N, 1), jnp.int32),
            jax.ShapeDtypeStruct((_N, 1), jnp.int32),
            jax.ShapeDtypeStruct((_NB, 1, _E), jnp.int32),
        ],
    )(x, gate_w, gate_b.reshape(1, _E))
    return outs


def _slot_body(i1_ref, i2_ref, r1_ref, r2_ref, base_ref, s1_ref, s2_ref,
               p1_ref, p2_ref, s1b_ref, s2b_ref):
    cols = lax.broadcasted_iota(jnp.int32, (_TB, _E), 1)
    base = base_ref[0]
    o1 = cols == i1_ref[...]
    o2 = cols == i2_ref[...]
    b1 = jnp.sum(jnp.where(o1, base, 0), axis=1, keepdims=True)
    b2 = jnp.sum(jnp.where(o2, base, 0), axis=1, keepdims=True)
    p1_ref[...] = b1 + r1_ref[...]
    p2_ref[...] = b2 + r2_ref[...]
    s1b_ref[...] = jnp.broadcast_to(s1_ref[...], (_TB, 16))
    s2b_ref[...] = jnp.broadcast_to(s2_ref[...], (_TB, 16))


def _slots(i1, i2, r1, r2, baseb, s1, s2):
    return pl.pallas_call(
        _slot_body,
        grid=(_NB,),
        in_specs=[pl.BlockSpec((_TB, 1), lambda t: (t, 0))] * 4
        + [pl.BlockSpec((1, 1, _E), lambda t: (t, 0, 0))]
        + [pl.BlockSpec((_TB, 1), lambda t: (t, 0))] * 2,
        out_specs=[pl.BlockSpec((_TB, 1), lambda t: (t, 0))] * 2
        + [pl.BlockSpec((_TB, 16), lambda t: (t, 0))] * 2,
        out_shape=[
            jax.ShapeDtypeStruct((_N, 1), jnp.int32),
            jax.ShapeDtypeStruct((_N, 1), jnp.int32),
            jax.ShapeDtypeStruct((_N, 16), jnp.float32),
            jax.ShapeDtypeStruct((_N, 16), jnp.float32),
        ],
    )(i1, i2, r1, r2, baseb, s1, s2)


def _mlp_body(eot_ref, live_ref, start_ref, run_ref, runs_e_ref, nr_ref,
              x_ref, w1_hbm, w2_hbm, b1_ref, b2_ref, out_ref,
              w1b, w2b, sem1, sem2):
    t = pl.program_id(0)
    r = run_ref[t]

    def _start_fetch(rf):
        slot = lax.rem(rf, 3)
        e = runs_e_ref[rf]
        pltpu.make_async_copy(w1_hbm.at[e], w1b.at[slot],
                              sem1.at[slot]).start()
        pltpu.make_async_copy(w2_hbm.at[e], w2b.at[slot],
                              sem2.at[slot]).start()

    @pl.when(t == 0)
    def _prologue():
        _start_fetch(0)

        @pl.when(nr_ref[0] > 1)
        def _p1():
            _start_fetch(1)

        @pl.when(nr_ref[0] > 2)
        def _p2():
            _start_fetch(2)

    @pl.when((start_ref[t] != 0) & (t > 0) & (r + 2 < nr_ref[0]))
    def _fetch_ahead():
        _start_fetch(r + 2)

    @pl.when(start_ref[t] != 0)
    def _wait():
        slot = lax.rem(r, 3)
        e = runs_e_ref[r]
        pltpu.make_async_copy(w1_hbm.at[e], w1b.at[slot],
                              sem1.at[slot]).wait()
        pltpu.make_async_copy(w2_hbm.at[e], w2b.at[slot],
                              sem2.at[slot]).wait()

    @pl.when(live_ref[t] != 0)
    def _compute():
        slot = lax.rem(r, 3)
        a = (jnp.dot(x_ref[...], w1b[slot],
                     preferred_element_type=jnp.float32) + b1_ref[0])
        h = 0.5 * a * (1.0 + lax.erf(a * 0.7071067811865476))
        out_ref[...] = (jnp.dot(h, w2b[slot],
                                preferred_element_type=jnp.float32)
                        + b2_ref[0])


def _expert_mlp(eot, live, is_start, run_of, runs_e, n_runs,
                x_padded, w1, w2, b1, b2):
    grid_spec = pltpu.PrefetchScalarGridSpec(
        num_scalar_prefetch=6,
        grid=(_NT,),
        in_specs=[
            pl.BlockSpec((_TM, _D), lambda t, *_: (t, 0)),
            pl.BlockSpec(memory_space=pltpu.MemorySpace.HBM),
            pl.BlockSpec(memory_space=pltpu.MemorySpace.HBM),
            pl.BlockSpec((1, 1, _DFF), lambda t, eot, *_: (eot[t], 0, 0)),
            pl.BlockSpec((1, 1, _D), lambda t, eot, *_: (eot[t], 0, 0)),
        ],
        out_specs=pl.BlockSpec((_TM, _D), lambda t, *_: (t, 0)),
        scratch_shapes=[
            pltpu.VMEM((3, _D, _DFF), jnp.float32),
            pltpu.VMEM((3, _DFF, _D), jnp.float32),
            pltpu.SemaphoreType.DMA((3,)),
            pltpu.SemaphoreType.DMA((3,)),
        ],
    )
    return pl.pallas_call(
        _mlp_body,
        grid_spec=grid_spec,
        out_shape=jax.ShapeDtypeStruct((_NP, _D), jnp.float32),
        compiler_params=pltpu.CompilerParams(
            dimension_semantics=("arbitrary",)),
    )(eot, live, is_start, run_of, runs_e, n_runs, x_padded, w1, w2,
      b1.reshape(_E, 1, _DFF), b2.reshape(_E, 1, _D))


_DCHUNK = 32
_NCH = (_N // _NW) // _DCHUNK
_CCH = 16
_NCC = (_N // _NW) // _CCH


def _sc_dispatch(x, p1, p2):
    toks_per_w = _N // _NW
    mesh = plsc.VectorSubcoreMesh(core_axis_name="c", subcore_axis_name="s")

    @functools.partial(
        pl.kernel, mesh=mesh,
        out_type=jax.ShapeDtypeStruct((_NP, _D), jnp.float32),
        scratch_types=[
            pltpu.VMEM((_NCH, _DCHUNK), jnp.int32),
            pltpu.VMEM((_NCH, _DCHUNK), jnp.int32),
            pltpu.VMEM((2, _DCHUNK, _D), jnp.float32),
            pltpu.SemaphoreType.DMA((2,)),
            pltpu.SemaphoreType.DMA,
            pltpu.SemaphoreType.DMA,
        ],
    )
    def _dispatch(x_hbm, p1_hbm, p2_hbm, out_hbm, i0_v, i1_v, xbuf,
                  sx, s0, s1):
        wid = lax.axis_index("s") * _NC + lax.axis_index("c")
        base = wid * toks_per_w
        pltpu.sync_copy(p1_hbm.at[wid], i0_v)
        pltpu.sync_copy(p2_hbm.at[wid], i1_v)

        def _xread(i, slot):
            return pltpu.make_async_copy(
                x_hbm.at[pl.ds(base + i * _DCHUNK, _DCHUNK)],
                xbuf.at[slot], sx.at[slot])

        _xread(0, 0).start()

        def body(i, _):
            slot = lax.rem(i, 2)
            _xread(i, slot).wait()

            @pl.when(i + 1 < _NCH)
            def _ahead():
                _xread(i + 1, 1 - slot).start()

            cp0 = pltpu.async_copy(xbuf.at[slot], out_hbm.at[i0_v.at[i]], s0)
            cp1 = pltpu.async_copy(xbuf.at[slot], out_hbm.at[i1_v.at[i]], s1)
            cp0.wait()
            cp1.wait()
            return _

        lax.fori_loop(0, _NCH, body, 0)

    return _dispatch(x, p1.reshape(_NW, _NCH, _DCHUNK),
                     p2.reshape(_NW, _NCH, _DCHUNK))


def _sc_combine(y, p1, p2, sc1, sc2):
    toks_per_w = _N // _NW
    mesh = plsc.VectorSubcoreMesh(core_axis_name="c", subcore_axis_name="s")

    @functools.partial(
        pl.kernel, mesh=mesh,
        out_type=jax.ShapeDtypeStruct((_N, _D), jnp.float32),
        scratch_types=[
            pltpu.VMEM((_NCC, _CCH), jnp.int32),
            pltpu.VMEM((_NCC, _CCH), jnp.int32),
            pltpu.VMEM((_NCC, _CCH, 16), jnp.float32),
            pltpu.VMEM((_NCC, _CCH, 16), jnp.float32),
            pltpu.VMEM((2, _CCH, _D), jnp.float32),
            pltpu.VMEM((2, _CCH, _D), jnp.float32),
            pltpu.SemaphoreType.DMA((2,)),
            pltpu.SemaphoreType.DMA((2,)),
        ],
    )
    def _combine(y_hbm, p1_hbm, p2_hbm, sc1_hbm, sc2_hbm, out_hbm,
                 i0_v, i1_v, s0_v, s1_v, a_v, b_v, sem0, sem1):
        wid = lax.axis_index("s") * _NC + lax.axis_index("c")
        base = wid * toks_per_w
        pltpu.sync_copy(p1_hbm.at[wid], i0_v)
        pltpu.sync_copy(p2_hbm.at[wid], i1_v)
        pltpu.sync_copy(sc1_hbm.at[wid], s0_v)
        pltpu.sync_copy(sc2_hbm.at[wid], s1_v)

        def _gathers(i, slot):
            return (pltpu.make_async_copy(y_hbm.at[i0_v.at[i]],
                                          a_v.at[slot], sem0.at[slot]),
                    pltpu.make_async_copy(y_hbm.at[i1_v.at[i]],
                                          b_v.at[slot], sem1.at[slot]))

        g0, g1 = _gathers(0, 0)
        g0.start()
        g1.start()

        def body(i, _):
            slot = lax.rem(i, 2)
            g0, g1 = _gathers(i, slot)
            g0.wait()
            g1.wait()

            @pl.when(i + 1 < _NCC)
            def _ahead():
                n0, n1 = _gathers(i + 1, 1 - slot)
                n0.start()
                n1.start()

            def row(r, _):
                sa = s0_v[i, r, pl.ds(0, 16)]
                sb = s1_v[i, r, pl.ds(0, 16)]
                for j in range(_D // 16):
                    sl = pl.ds(j * 16, 16)
                    a_v[slot, r, sl] = (sa * a_v[slot, r, sl]
                                        + sb * b_v[slot, r, sl])
                return _

            lax.fori_loop(0, _CCH, row, 0)
            pltpu.sync_copy(
                a_v.at[slot],
                out_hbm.at[pl.ds(base + i * _CCH, _CCH)])
            return _

        lax.fori_loop(0, _NCC, body, 0)

    return _combine(y, p1.reshape(_NW, _NCC, _CCH),
                    p2.reshape(_NW, _NCC, _CCH),
                    sc1.reshape(_NW, _NCC, _CCH, 16),
                    sc2.reshape(_NW, _NCC, _CCH, 16))


def kernel(x, gate_w, gate_b, w1, b1, w2, b2):
    i1, i2, s1, s2, r1, r2, hist = _route(x, gate_w, gate_b)

    h = hist.reshape(_NB, _E)
    bb = jnp.cumsum(h, axis=0) - h
    counts = h.sum(axis=0)
    tiles = (counts + _TM - 1) // _TM
    tile_ends = jnp.cumsum(tiles)
    pstart = _TM * (tile_ends - tiles)
    baseb = (bb + pstart[None, :]).astype(jnp.int32).reshape(_NB, 1, _E)
    tidx = jnp.arange(_NT)
    eot = jnp.clip(jnp.searchsorted(tile_ends, tidx, side='right'),
                   0, _E - 1).astype(jnp.int32)
    live = (tidx < tile_ends[-1]).astype(jnp.int32)
    is_start = jnp.concatenate(
        [jnp.ones((1,), jnp.int32),
         (eot[1:] != eot[:-1]).astype(jnp.int32)])
    run_of = jnp.cumsum(is_start) - 1
    runs_e = jnp.zeros((_NT,), jnp.int32).at[run_of].set(eot)
    n_runs = run_of[-1:] + 1
    run_of = run_of.astype(jnp.int32)
    n_runs = n_runs.astype(jnp.int32)

    p1, p2, s1b, s2b = _slots(i1, i2, r1, r2, baseb, s1, s2)
    p1 = p1.reshape(_N)
    p2 = p2.reshape(_N)

    x_padded = _sc_dispatch(x, p1, p2)
    y = _expert_mlp(eot, live, is_start, run_of, runs_e, n_runs,
                    x_padded, w1, w2, b1, b2)
    return _sc_combine(y, p1, p2, s1b, s2b)


# speedup vs baseline: 1.9825x; 1.0098x over previous
import functools

import jax
import jax.numpy as jnp
from jax import lax
from jax.experimental import pallas as pl
from jax.experimental.pallas import tpu as pltpu
from jax.experimental.pallas import tpu_sc as plsc

_E = 64
_TOPK = 2
_D = 768
_DFF = 1536
_N = 4096

_TM = 128
_NT = (_N * _TOPK) // _TM + _E
_NP = _NT * _TM

_TB = 1024
_NB = _N // _TB

_NC = 2
_NS = 16
_NW = _NC * _NS


def _router_body(x_ref, gw_ref, gb_ref, i1_ref, i2_ref, s1_ref, s2_ref,
                 r1_ref, r2_ref, hist_ref):
    logits = jnp.dot(x_ref[...], gw_ref[...],
                     preferred_element_type=jnp.float32) + gb_ref[...]
    cols = lax.broadcasted_iota(jnp.int32, logits.shape, 1)
    m1 = jnp.max(logits, axis=1, keepdims=True)
    i1 = jnp.min(jnp.where(logits == m1, cols, _E), axis=1, keepdims=True)
    masked = jnp.where(cols == i1, -jnp.inf, logits)
    m2 = jnp.max(masked, axis=1, keepdims=True)
    i2 = jnp.min(jnp.where(masked == m2, cols, _E), axis=1, keepdims=True)
    z = jnp.exp(m2 - m1)
    denom = 1.0 + z
    i1_ref[...] = i1
    i2_ref[...] = i2
    s1_ref[...] = 1.0 / denom
    s2_ref[...] = z / denom

    o1 = (cols == i1).astype(jnp.float32)
    o2 = (cols == i2).astype(jnp.float32)
    s = o1 + o2
    rows_i = lax.broadcasted_iota(jnp.int32, (_TB, _TB), 0)
    cols_i = lax.broadcasted_iota(jnp.int32, (_TB, _TB), 1)
    ltri = (cols_i < rows_i).astype(jnp.float32)
    c = jnp.dot(ltri, s, preferred_element_type=jnp.float32)
    r1_ref[...] = jnp.sum(c * o1, axis=1, keepdims=True).astype(jnp.int32)
    r2_ref[...] = jnp.sum(c * o2, axis=1, keepdims=True).astype(jnp.int32)
    hist_ref[...] = jnp.sum(s, axis=0, keepdims=True)[None].astype(jnp.int32)


def _route(x, gate_w, gate_b):
    outs = pl.pallas_call(
        _router_body,
        grid=(_NB,),
        in_specs=[
            pl.BlockSpec((_TB, _D), lambda t: (t, 0)),
            pl.BlockSpec((_D, _E), lambda t: (0, 0)),
            pl.BlockSpec((1, _E), lambda t: (0, 0)),
        ],
        out_specs=[pl.BlockSpec((_TB, 1), lambda t: (t, 0))] * 6
        + [pl.BlockSpec((1, 1, _E), lambda t: (t, 0, 0))],
        out_shape=[
            jax.ShapeDtypeStruct((_N, 1), jnp.int32),
            jax.ShapeDtypeStruct((_N, 1), jnp.int32),
            jax.ShapeDtypeStruct((_N, 1), jnp.float32),
            jax.ShapeDtypeStruct((_N, 1), jnp.float32),
            jax.ShapeDtypeStruct((_N, 1), jnp.int32),
            jax.ShapeDtypeStruct((_N, 1), jnp.int32),
            jax.ShapeDtypeStruct((_NB, 1, _E), jnp.int32),
        ],
    )(x, gate_w, gate_b.reshape(1, _E))
    return outs


def _slot_body(i1_ref, i2_ref, r1_ref, r2_ref, base_ref, s1_ref, s2_ref,
               p1_ref, p2_ref, s1b_ref, s2b_ref):
    cols = lax.broadcasted_iota(jnp.int32, (_TB, _E), 1)
    base = base_ref[0]
    o1 = cols == i1_ref[...]
    o2 = cols == i2_ref[...]
    b1 = jnp.sum(jnp.where(o1, base, 0), axis=1, keepdims=True)
    b2 = jnp.sum(jnp.where(o2, base, 0), axis=1, keepdims=True)
    p1_ref[...] = b1 + r1_ref[...]
    p2_ref[...] = b2 + r2_ref[...]
    s1b_ref[...] = jnp.broadcast_to(s1_ref[...], (_TB, 16))
    s2b_ref[...] = jnp.broadcast_to(s2_ref[...], (_TB, 16))


def _slots(i1, i2, r1, r2, baseb, s1, s2):
    return pl.pallas_call(
        _slot_body,
        grid=(_NB,),
        in_specs=[pl.BlockSpec((_TB, 1), lambda t: (t, 0))] * 4
        + [pl.BlockSpec((1, 1, _E), lambda t: (t, 0, 0))]
        + [pl.BlockSpec((_TB, 1), lambda t: (t, 0))] * 2,
        out_specs=[pl.BlockSpec((_TB, 1), lambda t: (t, 0))] * 2
        + [pl.BlockSpec((_TB, 16), lambda t: (t, 0))] * 2,
        out_shape=[
            jax.ShapeDtypeStruct((_N, 1), jnp.int32),
            jax.ShapeDtypeStruct((_N, 1), jnp.int32),
            jax.ShapeDtypeStruct((_N, 16), jnp.float32),
            jax.ShapeDtypeStruct((_N, 16), jnp.float32),
        ],
    )(i1, i2, r1, r2, baseb, s1, s2)


def _mlp_body(eot_ref, live_ref, start_ref, run_ref, runs_e_ref, nr_ref,
              x_ref, w1_hbm, w2_hbm, b1_ref, b2_ref, out_ref,
              w1b, w2b, sem1, sem2):
    t = pl.program_id(0)
    r = run_ref[t]

    def _start_fetch(rf):
        slot = lax.rem(rf, 3)
        e = runs_e_ref[rf]
        pltpu.make_async_copy(w1_hbm.at[e], w1b.at[slot],
                              sem1.at[slot]).start()
        pltpu.make_async_copy(w2_hbm.at[e], w2b.at[slot],
                              sem2.at[slot]).start()

    @pl.when(t == 0)
    def _prologue():
        _start_fetch(0)

        @pl.when(nr_ref[0] > 1)
        def _p1():
            _start_fetch(1)

        @pl.when(nr_ref[0] > 2)
        def _p2():
            _start_fetch(2)

    @pl.when((start_ref[t] != 0) & (t > 0) & (r + 2 < nr_ref[0]))
    def _fetch_ahead():
        _start_fetch(r + 2)

    @pl.when(start_ref[t] != 0)
    def _wait():
        slot = lax.rem(r, 3)
        e = runs_e_ref[r]
        pltpu.make_async_copy(w1_hbm.at[e], w1b.at[slot],
                              sem1.at[slot]).wait()
        pltpu.make_async_copy(w2_hbm.at[e], w2b.at[slot],
                              sem2.at[slot]).wait()

    @pl.when(live_ref[t] != 0)
    def _compute():
        slot = lax.rem(r, 3)
        a = (jnp.dot(x_ref[...], w1b[slot],
                     preferred_element_type=jnp.float32) + b1_ref[0])
        h = 0.5 * a * (1.0 + lax.erf(a * 0.7071067811865476))
        out_ref[...] = (jnp.dot(h, w2b[slot],
                                preferred_element_type=jnp.float32)
                        + b2_ref[0])


def _expert_mlp(eot, live, is_start, run_of, runs_e, n_runs,
                x_padded, w1, w2, b1, b2):
    grid_spec = pltpu.PrefetchScalarGridSpec(
        num_scalar_prefetch=6,
        grid=(_NT,),
        in_specs=[
            pl.BlockSpec((_TM, _D), lambda t, *_: (t, 0)),
            pl.BlockSpec(memory_space=pltpu.MemorySpace.HBM),
            pl.BlockSpec(memory_space=pltpu.MemorySpace.HBM),
            pl.BlockSpec((1, 1, _DFF), lambda t, eot, *_: (eot[t], 0, 0)),
            pl.BlockSpec((1, 1, _D), lambda t, eot, *_: (eot[t], 0, 0)),
        ],
        out_specs=pl.BlockSpec((_TM, _D), lambda t, *_: (t, 0)),
        scratch_shapes=[
            pltpu.VMEM((3, _D, _DFF), jnp.float32),
            pltpu.VMEM((3, _DFF, _D), jnp.float32),
            pltpu.SemaphoreType.DMA((3,)),
            pltpu.SemaphoreType.DMA((3,)),
        ],
    )
    return pl.pallas_call(
        _mlp_body,
        grid_spec=grid_spec,
        out_shape=jax.ShapeDtypeStruct((_NP, _D), jnp.float32),
        compiler_params=pltpu.CompilerParams(
            dimension_semantics=("arbitrary",)),
    )(eot, live, is_start, run_of, runs_e, n_runs, x_padded, w1, w2,
      b1.reshape(_E, 1, _DFF), b2.reshape(_E, 1, _D))


_DCHUNK = 32
_NCH = (_N // _NW) // _DCHUNK
_CCH = 16
_NCC = (_N // _NW) // _CCH


def _sc_dispatch(x, p1, p2):
    toks_per_w = _N // _NW
    mesh = plsc.VectorSubcoreMesh(core_axis_name="c", subcore_axis_name="s")

    @functools.partial(
        pl.kernel, mesh=mesh,
        out_type=jax.ShapeDtypeStruct((_NP, _D), jnp.float32),
        scratch_types=[
            pltpu.VMEM((_NCH, _DCHUNK), jnp.int32),
            pltpu.VMEM((_NCH, _DCHUNK), jnp.int32),
            pltpu.VMEM((2, _DCHUNK, _D), jnp.float32),
            pltpu.SemaphoreType.DMA((2,)),
            pltpu.SemaphoreType.DMA,
            pltpu.SemaphoreType.DMA,
        ],
    )
    def _dispatch(x_hbm, p1_hbm, p2_hbm, out_hbm, i0_v, i1_v, xbuf,
                  sx, s0, s1):
        wid = lax.axis_index("s") * _NC + lax.axis_index("c")
        base = wid * toks_per_w
        pltpu.sync_copy(p1_hbm.at[wid], i0_v)
        pltpu.sync_copy(p2_hbm.at[wid], i1_v)

        def _xread(i, slot):
            return pltpu.make_async_copy(
                x_hbm.at[pl.ds(base + i * _DCHUNK, _DCHUNK)],
                xbuf.at[slot], sx.at[slot])

        _xread(0, 0).start()

        def body(i, _):
            slot = lax.rem(i, 2)
            _xread(i, slot).wait()

            @pl.when(i + 1 < _NCH)
            def _ahead():
                _xread(i + 1, 1 - slot).start()

            cp0 = pltpu.async_copy(xbuf.at[slot], out_hbm.at[i0_v.at[i]], s0)
            cp1 = pltpu.async_copy(xbuf.at[slot], out_hbm.at[i1_v.at[i]], s1)
            cp0.wait()
            cp1.wait()
            return _

        lax.fori_loop(0, _NCH, body, 0)

    return _dispatch(x, p1.reshape(_NW, _NCH, _DCHUNK),
                     p2.reshape(_NW, _NCH, _DCHUNK))


def _sc_combine(y, p1, p2, sc1, sc2):
    toks_per_w = _N // _NW
    mesh = plsc.VectorSubcoreMesh(core_axis_name="c", subcore_axis_name="s")

    @functools.partial(
        pl.kernel, mesh=mesh,
        out_type=jax.ShapeDtypeStruct((_N, _D), jnp.float32),
        scratch_types=[
            pltpu.VMEM((_NCC, _CCH), jnp.int32),
            pltpu.VMEM((_NCC, _CCH), jnp.int32),
            pltpu.VMEM((_NCC, _CCH, 16), jnp.float32),
            pltpu.VMEM((_NCC, _CCH, 16), jnp.float32),
            pltpu.VMEM((2, _CCH, _D), jnp.float32),
            pltpu.VMEM((2, _CCH, _D), jnp.float32),
            pltpu.SemaphoreType.DMA((2,)),
            pltpu.SemaphoreType.DMA((2,)),
        ],
    )
    def _combine(y_hbm, p1_hbm, p2_hbm, sc1_hbm, sc2_hbm, out_hbm,
                 i0_v, i1_v, s0_v, s1_v, a_v, b_v, sem0, sem1):
        wid = lax.axis_index("s") * _NC + lax.axis_index("c")
        base = wid * toks_per_w
        pltpu.sync_copy(p1_hbm.at[wid], i0_v)
        pltpu.sync_copy(p2_hbm.at[wid], i1_v)
        pltpu.sync_copy(sc1_hbm.at[wid], s0_v)
        pltpu.sync_copy(sc2_hbm.at[wid], s1_v)

        def _gathers(i, slot):
            return (pltpu.make_async_copy(y_hbm.at[i0_v.at[i]],
                                          a_v.at[slot], sem0.at[slot]),
                    pltpu.make_async_copy(y_hbm.at[i1_v.at[i]],
                                          b_v.at[slot], sem1.at[slot]))

        g0, g1 = _gathers(0, 0)
        g0.start()
        g1.start()

        def body(i, _):
            slot = lax.rem(i, 2)
            g0, g1 = _gathers(i, slot)
            g0.wait()
            g1.wait()

            @pl.when(i + 1 < _NCC)
            def _ahead():
                n0, n1 = _gathers(i + 1, 1 - slot)
                n0.start()
                n1.start()

            def row(r, _):
                sa = s0_v[i, r, pl.ds(0, 16)]
                sb = s1_v[i, r, pl.ds(0, 16)]
                for j in range(_D // 16):
                    sl = pl.ds(j * 16, 16)
                    a_v[slot, r, sl] = (sa * a_v[slot, r, sl]
                                        + sb * b_v[slot, r, sl])
                return _

            lax.fori_loop(0, _CCH, row, 0)
            pltpu.sync_copy(
                a_v.at[slot],
                out_hbm.at[pl.ds(base + i * _CCH, _CCH)])
            return _

        lax.fori_loop(0, _NCC, body, 0)

    return _combine(y, p1.reshape(_NW, _NCC, _CCH),
                    p2.reshape(_NW, _NCC, _CCH),
                    sc1.reshape(_NW, _NCC, _CCH, 16),
                    sc2.reshape(_NW, _NCC, _CCH, 16))


def kernel(x, gate_w, gate_b, w1, b1, w2, b2):
    i1, i2, s1, s2, r1, r2, hist = _route(x, gate_w, gate_b)

    h = hist.reshape(_NB, _E)
    bb = jnp.cumsum(h, axis=0) - h
    counts = h.sum(axis=0)
    tiles = (counts + _TM - 1) // _TM
    tile_ends = jnp.cumsum(tiles)
    pstart = _TM * (tile_ends - tiles)
    baseb = (bb + pstart[None, :]).astype(jnp.int32).reshape(_NB, 1, _E)
    tidx = jnp.arange(_NT)
    eot = jnp.clip(jnp.searchsorted(tile_ends, tidx, side='right'),
                   0, _E - 1).astype(jnp.int32)
    live = (tidx < tile_ends[-1]).astype(jnp.int32)
    is_start = jnp.concatenate(
        [jnp.ones((1,), jnp.int32),
         (eot[1:] != eot[:-1]).astype(jnp.int32)])
    run_of = jnp.cumsum(is_start) - 1
    runs_e = jnp.zeros((_NT,), jnp.int32).at[run_of].set(eot)
    n_runs = run_of[-1:] + 1
    run_of = run_of.astype(jnp.int32)
    n_runs = n_runs.astype(jnp.int32)

    p1, p2, s1b, s2b = _slots(i1, i2, r1, r2, baseb, s1, s2)
    p1 = p1.reshape(_N)
    p2 = p2.reshape(_N)

    x_padded = _sc_dispatch(x, p1, p2)
    y = _expert_mlp(eot, live, is_start, run_of, runs_e, n_runs,
                    x_padded, w1, w2, b1, b2)
    return _sc_combine(y, p1, p2, s1b, s2b)
